# Initial kernel scaffold; baseline (speedup 1.0000x reference)
import jax
import jax.numpy as jnp
from jax.experimental import pallas as pl


def kernel(state_emb, phase_id, phase_embedding_weight, training):
    raise NotImplementedError("write your pallas kernel here")


# speedup vs baseline: 1.0110x; 1.0110x over previous
import jax
import jax.numpy as jnp
from jax.experimental import pallas as pl

_B, _L, _H = 4096, 200, 128
_N = 6
_BB = 32


def _add_phase_kernel(ids_ref, table_ref, state_ref, out_ref):
    ids = jnp.clip(ids_ref[...], 0, _N - 1)
    emb = jnp.zeros((_BB, _H), dtype=jnp.float32)
    for k in range(_N):
        row = table_ref[k : k + 1, :]
        emb = jnp.where(ids == k, row, emb)
    out_ref[...] = state_ref[...] + emb[:, None, :]


def kernel(state_emb, phase_id, phase_embedding_weight, training):
    del training
    nblk = _B // _BB
    ids2 = phase_id.reshape(_B, 1)
    return pl.pallas_call(
        _add_phase_kernel,
        grid=(nblk,),
        in_specs=[
            pl.BlockSpec((_BB, 1), lambda i: (i, 0)),
            pl.BlockSpec((_N, _H), lambda i: (0, 0)),
            pl.BlockSpec((_BB, _L, _H), lambda i: (i, 0, 0)),
        ],
        out_specs=pl.BlockSpec((_BB, _L, _H), lambda i: (i, 0, 0)),
        out_shape=jax.ShapeDtypeStruct((_B, _L, _H), jnp.float32),
    )(ids2, phase_embedding_weight, state_emb)
